# Initial kernel scaffold; baseline (speedup 1.0000x reference)
import jax
import jax.numpy as jnp
from jax.experimental import pallas as pl


def kernel(confidences, localizations, targets):
    raise NotImplementedError("write your pallas kernel here")


# speedup vs baseline: 8.3404x; 8.3404x over previous
import functools

import jax
import jax.numpy as jnp
from jax import lax
from jax.experimental import pallas as pl
from jax.experimental.pallas import tpu as pltpu
from jax.experimental.pallas import tpu_sc as plsc

ALPHA = 1.0
EPS = 1e-7
B = 32
N = 8732
C = 21
L = 16
NPAD = 8736
NCHUNK = 6
CH = NPAD // NCHUNK
GC = CH // L
LN2 = 0.6931471805599453


def _softlog(s):
    bits = lax.bitcast_convert_type(s, jnp.int32)
    e = jnp.float32(1.0) * (lax.shift_right_arithmetic(bits, 23) - 127)
    mbits = lax.bitwise_or(lax.bitwise_and(bits, 0x007FFFFF), 0x3F800000)
    m = lax.bitcast_convert_type(mbits, jnp.float32)
    z = (m - 1.0) / (m + 1.0)
    z2 = z * z
    p = z * (2.0 + z2 * (2.0 / 3.0 + z2 * (2.0 / 5.0 + z2 * (2.0 / 7.0 + z2 * (2.0 / 9.0)))))
    return e * LN2 + p


def _f32_key(v):
    b = lax.bitcast_convert_type(v, jnp.uint32)
    neg = lax.shift_right_logical(b, jnp.uint32(31)) > 0
    return jnp.where(neg, ~b, lax.bitwise_xor(b, jnp.uint32(0x80000000)))


def _key_f32(t):
    was_pos = lax.shift_right_logical(t, jnp.uint32(31)) > 0
    b = jnp.where(was_pos, lax.bitwise_xor(t, jnp.uint32(0x80000000)), ~t)
    return lax.bitcast_convert_type(b, jnp.float32)


def _sc_body(conf_hbm, tgt_hbm, loc_hbm, out_hbm,
             conf_v, tgt_v, loc_v, neg_v, key_v, out_v):
    w = lax.axis_index("s") * 2 + lax.axis_index("c")
    ar = lax.iota(jnp.int32, L)
    zero = jnp.zeros((L,), jnp.float32)

    def chunk_body(ci, accs):
        off = ci * CH
        pltpu.sync_copy(conf_hbm.at[pl.ds(w * (NPAD * C) + off * C, CH * C)], conf_v)
        pltpu.sync_copy(tgt_hbm.at[pl.ds(w * (NPAD * 5) + off * 5, CH * 5)], tgt_v)
        pltpu.sync_copy(loc_hbm.at[pl.ds(w * (NPAD * 4) + off * 4, CH * 4)], loc_v)

        def group_body(g, accs):
            pos_acc, nm_acc, loc_acc = accs
            base = g * L
            rows = base + ar
            valid = (off + rows) < N
            rC = rows * C
            r5 = rows * 5
            r4 = rows * 4

            xs = [plsc.load_gather(conf_v, [rC + c]) for c in range(C)]
            m = xs[0]
            for c in range(1, C):
                m = jnp.maximum(m, xs[c])
            s = zero
            for c in range(C):
                s = s + jnp.exp(xs[c] - m)
            lse = m + _softlog(s)

            lab_f = plsc.load_gather(tgt_v, [r5 + 4])
            lab = lab_f.astype(jnp.int32)
            pos = lab > 0
            safe_lab = jnp.where(pos, jnp.clip(lab, 0, C - 1), 0)
            x_lab = plsc.load_gather(conf_v, [rC + safe_lab])

            posv = jnp.logical_and(pos, valid)
            pos_acc = pos_acc + jnp.where(posv, lse - x_lab, 0.0)
            nm_acc = nm_acc + jnp.where(posv, 1.0, 0.0)

            ngv = jnp.where(jnp.logical_and(valid, jnp.logical_not(pos)),
                            lse - xs[0], jnp.float32(-1e30))
            neg_v[pl.ds(off + base, L)] = ngv
            key_v[pl.ds(off + base, L)] = _f32_key(ngv)

            x1, y1, x2, y2 = (plsc.load_gather(loc_v, [r4 + c]) for c in range(4))
            x1g, y1g, x2g, y2g = (plsc.load_gather(tgt_v, [r5 + c]) for c in range(4))
            xkis1 = jnp.maximum(x1, x1g)
            ykis1 = jnp.maximum(y1, y1g)
            xkis2 = jnp.minimum(x2, x2g)
            ykis2 = jnp.minimum(y2, y2g)
            imask = jnp.logical_and(ykis2 > ykis1, xkis2 > xkis1)
            intsctk = jnp.where(imask, (xkis2 - xkis1) * (ykis2 - ykis1), 0.0)
            unionk = (x2 - x1) * (y2 - y1) + (x2g - x1g) * (y2g - y1g) - intsctk
            iouk = intsctk / (unionk + EPS)
            area_c = (jnp.maximum(x2, x2g) - jnp.minimum(x1, x1g)) * \
                     (jnp.maximum(y2, y2g) - jnp.minimum(y1, y1g))
            miouk = iouk - (area_c - unionk) / (area_c + EPS)
            loc_acc = loc_acc + jnp.where(posv, 1.0 - miouk, 0.0)
            return pos_acc, nm_acc, loc_acc

        return lax.fori_loop(0, GC, group_body, accs)

    pos_acc, nm_acc, loc_acc = lax.fori_loop(
        0, NCHUNK, chunk_body, (zero, zero, zero))

    pos_loss = jnp.sum(pos_acc, axis=0)
    nm_f = jnp.sum(nm_acc, axis=0)
    loc_loss = jnp.sum(loc_acc, axis=0)
    nm = nm_f.astype(jnp.int32)
    k = jnp.minimum(3 * nm, N - nm)
    k_f = k.astype(jnp.float32)

    NG = NPAD // L

    def search_body(bi, t):
        cand = lax.bitwise_or(t, lax.shift_left(jnp.uint32(1), (31 - bi).astype(jnp.uint32)))
        cvec = jnp.full((L,), cand)

        def count_body(g, acc):
            kv = key_v[pl.ds(g * L, L)]
            return acc + jnp.where(kv >= cvec, 1.0, 0.0)

        cnt = jnp.sum(lax.fori_loop(0, NG, count_body, zero), axis=0)
        return jnp.where(cnt >= k_f, cand, t)

    t = lax.fori_loop(0, 32, search_body, jnp.uint32(0))
    tvec = jnp.full((L,), t)

    def tail_body(g, accs):
        cnt_acc, sum_acc = accs
        kv = key_v[pl.ds(g * L, L)]
        vv = neg_v[pl.ds(g * L, L)]
        gt = kv > tvec
        return (cnt_acc + jnp.where(gt, 1.0, 0.0),
                sum_acc + jnp.where(gt, vv, 0.0))

    cnt_gt, sum_gt = lax.fori_loop(0, NG, tail_body, (zero, zero))
    cnt_gt = jnp.sum(cnt_gt, axis=0)
    sum_gt = jnp.sum(sum_gt, axis=0)
    neg_loss = jnp.where(k > 0, sum_gt + (k_f - cnt_gt) * _key_f32(t), 0.0)

    total = pos_loss + neg_loss + ALPHA * loc_loss
    out_v[...] = jnp.where(ar == 0, total, jnp.where(ar == 1, nm_f, 0.0))
    pltpu.sync_copy(out_v, out_hbm.at[pl.ds(w * L, L)])


def _tc_combine_body(part_ref, o_ref):
    total = jnp.sum(part_ref[:, 0:1], keepdims=True)
    nh = jnp.sum(part_ref[:, 1:2], keepdims=True)
    o_ref[...] = jnp.where(nh == 0.0, jnp.float32(0.0),
                           total / jnp.maximum(nh, 1.0))


def kernel(confidences, localizations, targets):
    pad = NPAD - N
    conf_p = jnp.pad(confidences, ((0, 0), (0, pad), (0, 0))).reshape(B * NPAD * C)
    tgt_p = jnp.pad(targets, ((0, 0), (0, pad), (0, 0))).reshape(B * NPAD * 5)
    loc_p = jnp.pad(localizations, ((0, 0), (0, pad), (0, 0))).reshape(B * NPAD * 4)

    mesh = plsc.VectorSubcoreMesh(core_axis_name="c", subcore_axis_name="s",
                                  num_cores=2, num_subcores=16)
    parts = pl.kernel(
        _sc_body,
        out_type=jax.ShapeDtypeStruct((B * L,), jnp.float32),
        mesh=mesh,
        compiler_params=pltpu.CompilerParams(needs_layout_passes=False),
        scratch_types=[
            pltpu.VMEM((CH * C,), jnp.float32),
            pltpu.VMEM((CH * 5,), jnp.float32),
            pltpu.VMEM((CH * 4,), jnp.float32),
            pltpu.VMEM((NPAD,), jnp.float32),
            pltpu.VMEM((NPAD,), jnp.uint32),
            pltpu.VMEM((L,), jnp.float32),
        ],
    )(conf_p, tgt_p, loc_p)

    out = pl.pallas_call(
        _tc_combine_body,
        out_shape=jax.ShapeDtypeStruct((1, 1), jnp.float32),
    )(parts.reshape(B, L))
    return out[0, 0]
